# baseline (speedup 1.0000x reference)
import jax
import jax.numpy as jnp
from jax.experimental import pallas as pl

NUM_ANCHORS = 4
NUM_LABELS = 5
SPLIT = 4
BLK = 2048


def _fused_heads_kernel(*refs):
    x_refs = refs[:SPLIT]
    w_ref, b_ref, out_ref = refs[SPLIT:]
    w = w_ref[...]
    b = b_ref[...]
    for q in range(SPLIT):
        out_ref[q] = (
            jnp.dot(x_refs[q][...], w, preferred_element_type=jnp.float32) + b
        )


def kernel(hidden_states, W_conf, b_conf, W_cls, b_cls, W_reg, b_reg):
    bsz, seq_len, hidden = hidden_states.shape
    x = hidden_states.reshape(bsz * seq_len, hidden)
    n = bsz * seq_len
    rows_per_split = n // SPLIT
    steps = rows_per_split // BLK

    w = jnp.concatenate([W_conf, W_cls, W_reg], axis=1)
    b = jnp.concatenate([b_conf, b_cls, b_reg], axis=0).reshape(1, -1)
    c = w.shape[1]

    def x_spec(q):
        return pl.BlockSpec((BLK, hidden), lambda i, q=q: (q * steps + i, 0))

    out = pl.pallas_call(
        _fused_heads_kernel,
        grid=(steps,),
        in_specs=[x_spec(q) for q in range(SPLIT)]
        + [
            pl.BlockSpec((hidden, c), lambda i: (0, 0)),
            pl.BlockSpec((1, c), lambda i: (0, 0)),
        ],
        out_specs=pl.BlockSpec((SPLIT, BLK, c), lambda i: (0, i, 0)),
        out_shape=jax.ShapeDtypeStruct((SPLIT, rows_per_split, c), jnp.float32),
    )(*([x] * SPLIT), w, b)

    out = out.reshape(n, c)
    conf = out[:, :NUM_ANCHORS].reshape(bsz, seq_len, NUM_ANCHORS)
    cls_ = out[:, NUM_ANCHORS:NUM_ANCHORS + NUM_ANCHORS * NUM_LABELS].reshape(
        bsz, seq_len, NUM_ANCHORS, NUM_LABELS
    )
    reg = out[:, NUM_ANCHORS + NUM_ANCHORS * NUM_LABELS:].reshape(
        bsz, seq_len, NUM_ANCHORS, 2
    )
    return (conf, cls_, reg)


# speedup vs baseline: 1.1071x; 1.1071x over previous
import jax
import jax.numpy as jnp
from jax.experimental import pallas as pl

NUM_ANCHORS = 4
NUM_LABELS = 5
BLK = 4096


def _fused_heads_kernel(x_ref, wc_ref, bc_ref, wl_ref, bl_ref, wr_ref, br_ref,
                        conf_ref, cls_ref, reg_ref):
    x = x_ref[...]
    conf_ref[...] = (
        jnp.dot(x, wc_ref[...], preferred_element_type=jnp.float32) + bc_ref[...]
    )
    cls_ref[...] = (
        jnp.dot(x, wl_ref[...], preferred_element_type=jnp.float32) + bl_ref[...]
    )
    reg_ref[...] = (
        jnp.dot(x, wr_ref[...], preferred_element_type=jnp.float32) + br_ref[...]
    )


def kernel(hidden_states, W_conf, b_conf, W_cls, b_cls, W_reg, b_reg):
    bsz, seq_len, hidden = hidden_states.shape
    x = hidden_states.reshape(bsz * seq_len, hidden)
    n = bsz * seq_len
    nc, nl, nr = NUM_ANCHORS, NUM_ANCHORS * NUM_LABELS, NUM_ANCHORS * 2

    def const_spec(r, c):
        return pl.BlockSpec((r, c), lambda i: (0, 0))

    conf, cls_, reg = pl.pallas_call(
        _fused_heads_kernel,
        grid=(n // BLK,),
        in_specs=[
            pl.BlockSpec((BLK, hidden), lambda i: (i, 0)),
            const_spec(hidden, nc), const_spec(1, nc),
            const_spec(hidden, nl), const_spec(1, nl),
            const_spec(hidden, nr), const_spec(1, nr),
        ],
        out_specs=[
            pl.BlockSpec((BLK, nc), lambda i: (i, 0)),
            pl.BlockSpec((BLK, nl), lambda i: (i, 0)),
            pl.BlockSpec((BLK, nr), lambda i: (i, 0)),
        ],
        out_shape=[
            jax.ShapeDtypeStruct((n, nc), jnp.float32),
            jax.ShapeDtypeStruct((n, nl), jnp.float32),
            jax.ShapeDtypeStruct((n, nr), jnp.float32),
        ],
    )(x, W_conf, b_conf.reshape(1, nc), W_cls, b_cls.reshape(1, nl),
      W_reg, b_reg.reshape(1, nr))

    return (
        conf.reshape(bsz, seq_len, NUM_ANCHORS),
        cls_.reshape(bsz, seq_len, NUM_ANCHORS, NUM_LABELS),
        reg.reshape(bsz, seq_len, NUM_ANCHORS, 2),
    )
